# Initial kernel scaffold; baseline (speedup 1.0000x reference)
import jax
import jax.numpy as jnp
from jax.experimental import pallas as pl


def kernel(views_1, views_2, img, neg_idx):
    raise NotImplementedError("write your pallas kernel here")


# speedup vs baseline: 259.0437x; 259.0437x over previous
import functools

import jax
import jax.numpy as jnp
from jax import lax
from jax.experimental import pallas as pl
from jax.experimental.pallas import tpu as pltpu
from jax.experimental.pallas import tpu_sc as plsc

TEMPERATURE = 2.0
FACTOR = 0.8
NEG = 256
EPS = 1e-08

B = 4
C = 192
H = 24
W = 24
HW = H * W


def _prep_body(z_ref, v_ref, rgb_ref, ni_ref, a_ref, j_ref, s0_ref):
    z = z_ref[0]
    v = v_ref[0]
    g = lax.dot_general(z, v, (((1,), (0,)), ((), ())),
                        preferred_element_type=jnp.float32)
    n1sq = jnp.sum(z * z, axis=1)
    n2sq = jnp.sum(v * v, axis=0)
    n1m = lax.broadcast_in_dim(jnp.sqrt(n1sq), (HW, HW), (0,))
    n2m = lax.broadcast_in_dim(jnp.sqrt(n2sq), (HW, HW), (1,))
    den = jnp.maximum(n1m * n2m, EPS)

    pi = lax.broadcasted_iota(jnp.int32, (HW, HW), 0)
    qi = lax.broadcasted_iota(jnp.int32, (HW, HW), 1)
    dr = ((pi // W) - (qi // W)).astype(jnp.float32)
    dc = ((pi % W) - (qi % W)).astype(jnp.float32)
    diag = float(((H - 1) ** 2 + (W - 1) ** 2) ** 0.5)
    deuc = jnp.sqrt(dr * dr + dc * dc) * (1.0 / diag)

    acc = jnp.zeros((HW, HW), jnp.float32)
    for k in range(3):
        rk = rgb_ref[k]
        rp = lax.broadcast_in_dim(rk, (HW, HW), (0,))
        rq = lax.broadcast_in_dim(rk, (HW, HW), (1,))
        acc = acc + (rp - rq) * (rp - rq)
    drgb = jnp.sqrt(acc) * (1.0 / (3.0 ** 0.5))

    wgt = deuc * FACTOR + drgb * (1.0 - FACTOR)
    a_ref[0] = jnp.minimum(jnp.abs(g) * wgt / den, 1.0)
    j_ref[0] = ni_ref[0, 0] * W + ni_ref[0, 1]
    s0pix = jnp.minimum(n1sq / jnp.maximum(n1sq, EPS), 1.0)
    s0_ref[...] = jnp.full((1, 1, 128), jnp.sum(s0pix) * (1.0 / HW),
                           jnp.float32)


def _finish_body(p_ref, s0_ref, o_ref):
    parts = p_ref[...]
    negsum = parts[:, 0, :]
    for wkr in range(1, parts.shape[1]):
        negsum = negsum + parts[:, wkr, :]
    sim = negsum * (1.0 / (HW * TEMPERATURE))
    s0 = s0_ref[...][:, 0, 0:1]
    logp0 = jnp.clip(jnp.log(s0), -100.0, None)
    log1m = jnp.clip(jnp.log(1.0 - sim), -100.0, None)
    bce = -(logp0 + jnp.sum(log1m, axis=1, keepdims=True)) * (1.0 / (NEG + 1))
    loss = jnp.sum(bce) * (1.0 / B)
    out2 = jnp.sum(s0) * (1.0 / B)
    out3 = jnp.sum(sim) * (TEMPERATURE / (NEG * B))
    lanes = lax.broadcasted_iota(jnp.int32, (8, 128), 1)
    res = jnp.where(lanes == 0, loss,
                    jnp.where(lanes == 1, out2,
                              jnp.where(lanes == 2, out3, 0.0)))
    o_ref[...] = res


def _make_sc_gather(nc, nw, ppw):
    mesh = plsc.VectorSubcoreMesh(core_axis_name="c", subcore_axis_name="s")

    @functools.partial(
        pl.kernel,
        out_type=jax.ShapeDtypeStruct((B, nw, NEG), jnp.float32),
        mesh=mesh,
        scratch_types=[
            pltpu.VMEM((ppw, HW), jnp.float32),
            pltpu.VMEM((ppw, NEG), jnp.int32),
            pltpu.VMEM((NEG,), jnp.float32),
        ],
        compiler_params=pltpu.CompilerParams(use_tc_tiling_on_sc=False,
                                             needs_layout_passes=False),
    )
    def sc_gather(a_hbm, j_hbm, out_hbm, rows_v, idx_v, acc_v):
        cid = lax.axis_index("c")
        sid = lax.axis_index("s")
        wid = sid * nc + cid
        base = wid * ppw
        lane = lax.iota(jnp.int32, 16)
        for b in range(B):
            pltpu.sync_copy(a_hbm.at[b, pl.ds(base, ppw)], rows_v)
            pltpu.sync_copy(j_hbm.at[b, pl.ds(base, ppw)], idx_v)

            def tbody(t, accs):
                tv = jnp.full((16,), t, jnp.int32)
                new = []
                for i in range(NEG // 16):
                    col = lane + (i * 16)
                    jv = plsc.load_gather(idx_v, [tv, col])
                    gv = plsc.load_gather(rows_v, [tv, jv])
                    new.append(accs[i] + gv)
                return tuple(new)

            accs = lax.fori_loop(
                0, ppw, tbody,
                tuple(jnp.zeros((16,), jnp.float32) for _ in range(NEG // 16)))
            for i in range(NEG // 16):
                acc_v[pl.ds(i * 16, 16)] = accs[i]
            pltpu.sync_copy(acc_v, out_hbm.at[b, wid])

    return sc_gather


@jax.jit
def kernel(views_1, views_2, img, neg_idx):
    z1t = views_1.reshape(B, C, HW).transpose(0, 2, 1)
    v2 = views_2.reshape(B, C, HW)
    rgb = img[0].reshape(3, HW)
    ni = neg_idx

    a_mat, jflat, s0 = pl.pallas_call(
        _prep_body,
        grid=(B,),
        in_specs=[
            pl.BlockSpec((1, HW, C), lambda b: (b, 0, 0)),
            pl.BlockSpec((1, C, HW), lambda b: (b, 0, 0)),
            pl.BlockSpec((3, HW), lambda b: (0, 0)),
            pl.BlockSpec((1, 2, HW, NEG), lambda b: (b, 0, 0, 0)),
        ],
        out_specs=[
            pl.BlockSpec((1, HW, HW), lambda b: (b, 0, 0)),
            pl.BlockSpec((1, HW, NEG), lambda b: (b, 0, 0)),
            pl.BlockSpec((1, 1, 128), lambda b: (b, 0, 0)),
        ],
        out_shape=[
            jax.ShapeDtypeStruct((B, HW, HW), jnp.float32),
            jax.ShapeDtypeStruct((B, HW, NEG), jnp.int32),
            jax.ShapeDtypeStruct((B, 1, 128), jnp.float32),
        ],
    )(z1t, v2, rgb, ni)

    info = plsc.get_sparse_core_info()
    nw = info.num_cores * info.num_subcores
    ppw = HW // nw
    partials = _make_sc_gather(info.num_cores, nw, ppw)(a_mat, jflat)

    res = pl.pallas_call(
        _finish_body,
        in_specs=[
            pl.BlockSpec((B, nw, NEG), lambda: (0, 0, 0)),
            pl.BlockSpec((B, 1, 128), lambda: (0, 0, 0)),
        ],
        out_specs=pl.BlockSpec((8, 128), lambda: (0, 0)),
        out_shape=jax.ShapeDtypeStruct((8, 128), jnp.float32),
    )(partials, s0)

    return res[0, 0], res[0, 1], res[0, 2]


# speedup vs baseline: 265.0963x; 1.0234x over previous
import functools

import jax
import jax.numpy as jnp
from jax import lax
from jax.experimental import pallas as pl
from jax.experimental.pallas import tpu as pltpu
from jax.experimental.pallas import tpu_sc as plsc

TEMPERATURE = 2.0
FACTOR = 0.8
NEG = 256
EPS = 1e-08

B = 4
C = 192
H = 24
W = 24
HW = H * W


def _prep_body(z_ref, v_ref, rgb_ref, ni_ref, a_ref, j_ref, s0_ref, w_scr):
    b = pl.program_id(0)

    @pl.when(b == 0)
    def _():
        pi = lax.broadcasted_iota(jnp.int32, (HW, HW), 0)
        qi = lax.broadcasted_iota(jnp.int32, (HW, HW), 1)
        dr = ((pi // W) - (qi // W)).astype(jnp.float32)
        dc = ((pi % W) - (qi % W)).astype(jnp.float32)
        diag = float(((H - 1) ** 2 + (W - 1) ** 2) ** 0.5)
        deuc = jnp.sqrt(dr * dr + dc * dc) * (1.0 / diag)
        acc = jnp.zeros((HW, HW), jnp.float32)
        for k in range(3):
            rk = rgb_ref[k]
            rp = lax.broadcast_in_dim(rk, (HW, HW), (0,))
            rq = lax.broadcast_in_dim(rk, (HW, HW), (1,))
            acc = acc + (rp - rq) * (rp - rq)
        drgb = jnp.sqrt(acc) * (1.0 / (3.0 ** 0.5))
        w_scr[...] = deuc * FACTOR + drgb * (1.0 - FACTOR)

    z = z_ref[0]
    v = v_ref[0]
    g = lax.dot_general(z, v, (((0,), (0,)), ((), ())),
                        preferred_element_type=jnp.float32)
    n1sq = jnp.sum(z * z, axis=0)
    n2sq = jnp.sum(v * v, axis=0)
    n1m = lax.broadcast_in_dim(jnp.sqrt(n1sq), (HW, HW), (0,))
    n2m = lax.broadcast_in_dim(jnp.sqrt(n2sq), (HW, HW), (1,))
    den = jnp.maximum(n1m * n2m, EPS)
    a_ref[0] = jnp.minimum(jnp.abs(g) * w_scr[...] / den, 1.0)
    j_ref[0] = ni_ref[0, 0] * W + ni_ref[0, 1]
    s0pix = jnp.minimum(n1sq / jnp.maximum(n1sq, EPS), 1.0)
    s0_ref[...] = jnp.full((1, 1, 128), jnp.sum(s0pix) * (1.0 / HW),
                           jnp.float32)


def _finish_body(p_ref, s0_ref, o_ref):
    parts = p_ref[...]
    negsum = parts[:, 0, :]
    for wkr in range(1, parts.shape[1]):
        negsum = negsum + parts[:, wkr, :]
    sim = negsum * (1.0 / (HW * TEMPERATURE))
    s0 = s0_ref[...][:, 0, 0:1]
    logp0 = jnp.clip(jnp.log(s0), -100.0, None)
    log1m = jnp.clip(jnp.log(1.0 - sim), -100.0, None)
    bce = -(logp0 + jnp.sum(log1m, axis=1, keepdims=True)) * (1.0 / (NEG + 1))
    loss = jnp.sum(bce) * (1.0 / B)
    out2 = jnp.sum(s0) * (1.0 / B)
    out3 = jnp.sum(sim) * (TEMPERATURE / (NEG * B))
    lanes = lax.broadcasted_iota(jnp.int32, (8, 128), 1)
    res = jnp.where(lanes == 0, loss,
                    jnp.where(lanes == 1, out2,
                              jnp.where(lanes == 2, out3, 0.0)))
    o_ref[...] = res


def _make_sc_gather(nc, nw, ppw):
    mesh = plsc.VectorSubcoreMesh(core_axis_name="c", subcore_axis_name="s")

    @functools.partial(
        pl.kernel,
        out_type=jax.ShapeDtypeStruct((B, nw, NEG), jnp.float32),
        mesh=mesh,
        scratch_types=[
            pltpu.VMEM((ppw, HW), jnp.float32),
            pltpu.VMEM((ppw, NEG), jnp.int32),
            pltpu.VMEM((NEG,), jnp.float32),
        ],
        compiler_params=pltpu.CompilerParams(use_tc_tiling_on_sc=False,
                                             needs_layout_passes=False),
    )
    def sc_gather(a_hbm, j_hbm, out_hbm, rows_v, idx_v, acc_v):
        cid = lax.axis_index("c")
        sid = lax.axis_index("s")
        wid = sid * nc + cid
        base = wid * ppw
        lane = lax.iota(jnp.int32, 16)
        for b in range(B):
            pltpu.sync_copy(a_hbm.at[b, pl.ds(base, ppw)], rows_v)
            pltpu.sync_copy(j_hbm.at[b, pl.ds(base, ppw)], idx_v)

            def tbody(t, accs):
                tv = jnp.full((16,), t, jnp.int32)
                new = []
                for i in range(NEG // 16):
                    col = lane + (i * 16)
                    jv = plsc.load_gather(idx_v, [tv, col])
                    gv = plsc.load_gather(rows_v, [tv, jv])
                    new.append(accs[i] + gv)
                return tuple(new)

            accs = lax.fori_loop(
                0, ppw, tbody,
                tuple(jnp.zeros((16,), jnp.float32) for _ in range(NEG // 16)))
            for i in range(NEG // 16):
                acc_v[pl.ds(i * 16, 16)] = accs[i]
            pltpu.sync_copy(acc_v, out_hbm.at[b, wid])

    return sc_gather


def _run_prep(z1, v2, rgb, ni, interpret=False):
    return pl.pallas_call(
        _prep_body,
        grid=(B,),
        in_specs=[
            pl.BlockSpec((1, C, HW), lambda b: (b, 0, 0)),
            pl.BlockSpec((1, C, HW), lambda b: (b, 0, 0)),
            pl.BlockSpec((3, HW), lambda b: (0, 0)),
            pl.BlockSpec((1, 2, HW, NEG), lambda b: (b, 0, 0, 0)),
        ],
        out_specs=[
            pl.BlockSpec((1, HW, HW), lambda b: (b, 0, 0)),
            pl.BlockSpec((1, HW, NEG), lambda b: (b, 0, 0)),
            pl.BlockSpec((1, 1, 128), lambda b: (b, 0, 0)),
        ],
        out_shape=[
            jax.ShapeDtypeStruct((B, HW, HW), jnp.float32),
            jax.ShapeDtypeStruct((B, HW, NEG), jnp.int32),
            jax.ShapeDtypeStruct((B, 1, 128), jnp.float32),
        ],
        scratch_shapes=[pltpu.VMEM((HW, HW), jnp.float32)],
        interpret=interpret,
    )(z1, v2, rgb, ni)


def _run_finish(partials, s0, nw, interpret=False):
    return pl.pallas_call(
        _finish_body,
        in_specs=[
            pl.BlockSpec((B, nw, NEG), lambda: (0, 0, 0)),
            pl.BlockSpec((B, 1, 128), lambda: (0, 0, 0)),
        ],
        out_specs=pl.BlockSpec((8, 128), lambda: (0, 0)),
        out_shape=jax.ShapeDtypeStruct((8, 128), jnp.float32),
        interpret=interpret,
    )(partials, s0)


@jax.jit
def kernel(views_1, views_2, img, neg_idx):
    z1 = views_1.reshape(B, C, HW)
    v2 = views_2.reshape(B, C, HW)
    rgb = img[0].reshape(3, HW)

    a_mat, jflat, s0 = _run_prep(z1, v2, rgb, neg_idx)

    info = plsc.get_sparse_core_info()
    nw = info.num_cores * info.num_subcores
    ppw = HW // nw
    partials = _make_sc_gather(info.num_cores, nw, ppw)(a_mat, jflat)

    res = _run_finish(partials, s0, nw)
    return res[0, 0], res[0, 1], res[0, 2]
